# baseline (speedup 1.0000x reference)
import functools

import jax
import jax.numpy as jnp
from jax.experimental import pallas as pl
import jax.experimental.pallas.tpu as pltpu

_TT = 512
_KB = 1024
_HI = jax.lax.Precision.HIGHEST


def _stage_body(enc_ref, e2_ref, cb2_ref, cbs_ref, c2_ref,
                idx_ref, zq_ref, dist_ref, *, K, C, Tt, Kb):
    enc = enc_ref[...]
    e2 = e2_ref[...]
    dmin = jnp.full((Tt, 1), jnp.inf, jnp.float32)
    for kc in range(K // Kb):
        sl = pl.ds(kc * Kb, Kb)
        m2 = jax.lax.dot_general(
            enc, cb2_ref[:, sl], (((1,), (0,)), ((), ())),
            preferred_element_type=jnp.float32)
        dist = (e2 - m2) + c2_ref[:, sl]
        dist_ref[:, sl] = dist
        dmin = jnp.minimum(dmin, jnp.min(dist, axis=1, keepdims=True))
    big = jnp.int32(2 ** 30)
    idxi = jnp.full((Tt, 1), big)
    for kc in range(K // Kb):
        sl = pl.ds(kc * Kb, Kb)
        iota = jax.lax.broadcasted_iota(jnp.int32, (1, Kb), 1) + (kc * Kb)
        cand = jnp.where(dist_ref[:, sl] == dmin, iota, big)
        idxi = jnp.minimum(idxi, jnp.min(cand, axis=1, keepdims=True))
    idx_ref[...] = idxi
    sel = jnp.zeros((C, Tt), jnp.float32)
    for kc in range(K // Kb):
        sl = pl.ds(kc * Kb, Kb)
        iota = jax.lax.broadcasted_iota(jnp.int32, (1, Kb), 1) + (kc * Kb)
        m01 = (iota == idxi).astype(jnp.float32)
        sel = sel + jax.lax.dot_general(
            cbs_ref[:, sl], m01, (((1,), (1,)), ((), ())),
            preferred_element_type=jnp.float32,
            precision=_HI)
    zq_ref[0] = sel


def _vq_stage(enc, e2, cb):
    BT, C = enc.shape
    K = cb.shape[0]
    Tt, Kb = _TT, _KB
    nT = BT // Tt
    cb2 = 2.0 * cb.T
    cbs = cb.T
    c2 = (cb ** 2).sum(1)[None, :]
    body = functools.partial(_stage_body, K=K, C=C, Tt=Tt, Kb=Kb)
    idx, zq = pl.pallas_call(
        body,
        grid=(nT,),
        in_specs=[
            pl.BlockSpec((Tt, C), lambda t: (t, 0)),
            pl.BlockSpec((Tt, 1), lambda t: (t, 0)),
            pl.BlockSpec((C, K), lambda t: (0, 0)),
            pl.BlockSpec((C, K), lambda t: (0, 0)),
            pl.BlockSpec((1, K), lambda t: (0, 0)),
        ],
        out_specs=[
            pl.BlockSpec((Tt, 1), lambda t: (t, 0)),
            pl.BlockSpec((1, C, Tt), lambda t: (t, 0, 0)),
        ],
        out_shape=[
            jax.ShapeDtypeStruct((BT, 1), jnp.int32),
            jax.ShapeDtypeStruct((nT, C, Tt), jnp.float32),
        ],
        scratch_shapes=[pltpu.VMEM((Tt, K), jnp.float32)],
    )(enc, e2, cb2, cbs, c2)
    zq = jnp.transpose(zq, (1, 0, 2)).reshape(C, BT)
    return idx[:, 0], zq


def kernel(z, input_length, n_quantizers, W_in, b_in, W_out, b_out,
           qin_w, qin_b, qout_w, qout_b, codebooks):
    del input_length, n_quantizers
    z = z.astype(jnp.float32)
    B, D, T = z.shape
    Nq, K, C = codebooks.shape

    zp = jnp.einsum('od,bdt->bot', W_in, z) + b_in[None, :, None]
    residual = zp
    quantized_out = jnp.zeros_like(zp)
    codes = []
    commit_losses = []
    for i in range(Nq):
        z_e = (jnp.einsum('cd,bdt->bct', qin_w[i], residual)
               + qin_b[i][None, :, None])
        enc = jnp.transpose(z_e, (0, 2, 1)).reshape(-1, C)
        e2 = (enc ** 2).sum(1, keepdims=True)
        idx, zq = _vq_stage(enc, e2, codebooks[i])
        z_q = zq.reshape(C, B, T).transpose(1, 0, 2)
        commit = ((z_e - z_q) ** 2).mean(axis=(1, 2))
        z_q_st = z_e + jax.lax.stop_gradient(z_q - z_e)
        out_q = (jnp.einsum('dc,bct->bdt', qout_w[i], z_q_st)
                 + qout_b[i][None, :, None])
        quantized_out = quantized_out + out_q
        residual = residual - out_q
        codes.append(idx.reshape(B, T))
        commit_losses.append(commit)
    out = jnp.einsum('od,bdt->bot', W_out, quantized_out) + b_out[None, :, None]
    return out, jnp.stack(commit_losses), jnp.stack(codes)


# speedup vs baseline: 1.0068x; 1.0068x over previous
import functools

import jax
import jax.numpy as jnp
from jax.experimental import pallas as pl
import jax.experimental.pallas.tpu as pltpu

_TT = 512
_KB = 1024
_HI = jax.lax.Precision.HIGHEST


def _stage_body(enc_ref, cb2_ref, cbs_ref, c2_ref,
                idx_ref, zq_ref, dist_ref, *, K, C, Tt, Kb):
    enc = enc_ref[...]
    e2 = jnp.sum(enc * enc, axis=1, keepdims=True)
    dmin = jnp.full((Tt, 1), jnp.inf, jnp.float32)
    for kc in range(K // Kb):
        sl = pl.ds(kc * Kb, Kb)
        m2 = jax.lax.dot_general(
            enc, cb2_ref[:, sl], (((1,), (0,)), ((), ())),
            preferred_element_type=jnp.float32)
        dist = (e2 - m2) + c2_ref[:, sl]
        dist_ref[:, sl] = dist
        dmin = jnp.minimum(dmin, jnp.min(dist, axis=1, keepdims=True))
    big = jnp.int32(2 ** 30)
    idxi = jnp.full((Tt, 1), big)
    for kc in range(K // Kb):
        sl = pl.ds(kc * Kb, Kb)
        iota = jax.lax.broadcasted_iota(jnp.int32, (1, Kb), 1) + (kc * Kb)
        cand = jnp.where(dist_ref[:, sl] == dmin, iota, big)
        idxi = jnp.minimum(idxi, jnp.min(cand, axis=1, keepdims=True))
    idx_ref[...] = idxi
    sel = jnp.zeros((C, Tt), jnp.float32)
    for kc in range(K // Kb):
        sl = pl.ds(kc * Kb, Kb)
        iota = jax.lax.broadcasted_iota(jnp.int32, (1, Kb), 1) + (kc * Kb)
        m01 = (iota == idxi).astype(jnp.float32)
        sel = sel + jax.lax.dot_general(
            cbs_ref[:, sl], m01, (((1,), (1,)), ((), ())),
            preferred_element_type=jnp.float32,
            precision=_HI)
    zq_ref[0] = sel


def _vq_stage(enc, cb):
    BT, C = enc.shape
    K = cb.shape[0]
    Tt, Kb = _TT, _KB
    nT = BT // Tt
    cb2 = 2.0 * cb.T
    cbs = cb.T
    c2 = (cb ** 2).sum(1)[None, :]
    body = functools.partial(_stage_body, K=K, C=C, Tt=Tt, Kb=Kb)
    idx, zq = pl.pallas_call(
        body,
        grid=(nT,),
        in_specs=[
            pl.BlockSpec((Tt, C), lambda t: (t, 0)),
            pl.BlockSpec((C, K), lambda t: (0, 0)),
            pl.BlockSpec((C, K), lambda t: (0, 0)),
            pl.BlockSpec((1, K), lambda t: (0, 0)),
        ],
        out_specs=[
            pl.BlockSpec((Tt, 1), lambda t: (t, 0)),
            pl.BlockSpec((1, C, Tt), lambda t: (t, 0, 0)),
        ],
        out_shape=[
            jax.ShapeDtypeStruct((BT, 1), jnp.int32),
            jax.ShapeDtypeStruct((nT, C, Tt), jnp.float32),
        ],
        scratch_shapes=[pltpu.VMEM((Tt, K), jnp.float32)],
    )(enc, cb2, cbs, c2)
    zq = jnp.transpose(zq, (1, 0, 2)).reshape(C, BT)
    return idx[:, 0], zq


def kernel(z, input_length, n_quantizers, W_in, b_in, W_out, b_out,
           qin_w, qin_b, qout_w, qout_b, codebooks):
    del input_length, n_quantizers
    z = z.astype(jnp.float32)
    B, D, T = z.shape
    Nq, K, C = codebooks.shape

    zp = jnp.einsum('od,bdt->bot', W_in, z) + b_in[None, :, None]
    residual = zp
    quantized_out = jnp.zeros_like(zp)
    codes = []
    commit_losses = []
    for i in range(Nq):
        z_e = (jnp.einsum('cd,bdt->bct', qin_w[i], residual)
               + qin_b[i][None, :, None])
        enc = jnp.transpose(z_e, (0, 2, 1)).reshape(-1, C)
        idx, zq = _vq_stage(enc, codebooks[i])
        z_q = zq.reshape(C, B, T).transpose(1, 0, 2)
        commit = ((z_e - z_q) ** 2).mean(axis=(1, 2))
        z_q_st = z_e + jax.lax.stop_gradient(z_q - z_e)
        out_q = (jnp.einsum('dc,bct->bdt', qout_w[i], z_q_st)
                 + qout_b[i][None, :, None])
        quantized_out = quantized_out + out_q
        residual = residual - out_q
        codes.append(idx.reshape(B, T))
        commit_losses.append(commit)
    out = jnp.einsum('od,bdt->bot', W_out, quantized_out) + b_out[None, :, None]
    return out, jnp.stack(commit_losses), jnp.stack(codes)
